# Initial kernel scaffold; baseline (speedup 1.0000x reference)
import jax
import jax.numpy as jnp
from jax.experimental import pallas as pl


def kernel(x, edge_index, fc1_W, fc1_b, W1, b1, g1, be1, W2, b2, g2, be2, W3, b3, g3, be3, fc2_W, fc2_b):
    raise NotImplementedError("write your pallas kernel here")


# speedup vs baseline: 9.5905x; 9.5905x over previous
import functools

import jax
import jax.numpy as jnp
from jax import lax
from jax.experimental import pallas as pl
from jax.experimental.pallas import tpu as pltpu
from jax.experimental.pallas import tpu_sc as plsc

_NC = 2
_NS = 16
_NW = _NC * _NS
_B = 128


def _mesh():
    return plsc.VectorSubcoreMesh(core_axis_name="c", subcore_axis_name="s")


_FD = 128


def _make_deg_kernel(nr, nb):
    slab = nr // _NS

    @functools.partial(
        pl.kernel,
        out_type=jax.ShapeDtypeStruct((_NC, nr, _FD), jnp.float32),
        mesh=_mesh(),
        scratch_types=[
            pltpu.VMEM((nb, _B), jnp.int32),
            pltpu.VMEM((_B, _FD), jnp.float32),
            pltpu.VMEM_SHARED((nr, _FD), jnp.float32),
        ],
    )
    def deg_kernel(dst_hbm, zeros_hbm, ones_hbm, out_hbm, dst_v, ones_v, deg_sh):
        cid = lax.axis_index("c")
        sid = lax.axis_index("s")
        wid = cid * _NS + sid
        pltpu.sync_copy(ones_hbm, ones_v)
        pltpu.sync_copy(dst_hbm.at[wid], dst_v)
        pltpu.sync_copy(zeros_hbm, deg_sh.at[pl.ds(sid * slab, slab)])
        plsc.subcore_barrier()

        def body(j, carry):
            pltpu.sync_copy(ones_v, deg_sh.at[dst_v.at[j]], add=True)
            return carry

        lax.fori_loop(0, nb, body, 0)
        plsc.subcore_barrier()
        pltpu.sync_copy(deg_sh.at[pl.ds(sid * slab, slab)],
                        out_hbm.at[cid, pl.ds(sid * slab, slab)])

    return deg_kernel


def _make_scat_kernel(nr, nb, f):
    slab = nr // _NS

    @functools.partial(
        pl.kernel,
        out_type=jax.ShapeDtypeStruct((_NC, nr, f), jnp.float32),
        mesh=_mesh(),
        scratch_types=[
            pltpu.VMEM((nb, _B), jnp.int32),
            pltpu.VMEM((nb, _B), jnp.int32),
            pltpu.VMEM((_B, f), jnp.float32),
            pltpu.VMEM_SHARED((nr, f), jnp.float32),
            pltpu.SemaphoreType.DMA,
        ],
    )
    def scat_kernel(s_hbm, src_hbm, dst_hbm, zeros_hbm, out_hbm,
                    src_v, dst_v, rows_v, acc_sh, gsem):
        cid = lax.axis_index("c")
        sid = lax.axis_index("s")
        wid = cid * _NS + sid
        pltpu.sync_copy(src_hbm.at[wid], src_v)
        pltpu.sync_copy(dst_hbm.at[wid], dst_v)
        pltpu.sync_copy(zeros_hbm, acc_sh.at[pl.ds(sid * slab, slab)])
        plsc.subcore_barrier()

        def body(j, carry):
            pltpu.async_copy(s_hbm.at[src_v.at[j]], rows_v, gsem).wait()
            pltpu.sync_copy(rows_v, acc_sh.at[dst_v.at[j]], add=True)
            return carry

        lax.fori_loop(0, nb, body, 0)
        plsc.subcore_barrier()
        pltpu.sync_copy(acc_sh.at[pl.ds(sid * slab, slab)],
                        out_hbm.at[cid, pl.ds(sid * slab, slab)])

    return scat_kernel


def _stage_a(x_ref, w_ref, b_ref, w1_ref, dinv_ref, s1_ref):
    h = jnp.dot(x_ref[...], w_ref[...], preferred_element_type=jnp.float32)
    h = jnp.maximum(h + b_ref[...], 0.0)
    t = jnp.dot(h, w1_ref[...], preferred_element_type=jnp.float32)
    s1_ref[...] = t * dinv_ref[...]


def _stage_mid(acc_ref, s_ref, dinv_ref, b_ref, g_ref, be_ref, w_ref, out_ref, *, n):
    a = acc_ref[0, :n, :] + acc_ref[1, :n, :]
    g = (a + s_ref[...]) * dinv_ref[...] + b_ref[...]
    m = jnp.mean(g, axis=0, keepdims=True)
    cv = g - m
    v = jnp.mean(cv * cv, axis=0, keepdims=True)
    r = jnp.maximum(cv * lax.rsqrt(v + 1e-5) * g_ref[...] + be_ref[...], 0.0)
    t = jnp.dot(r, w_ref[...], preferred_element_type=jnp.float32)
    out_ref[...] = t * dinv_ref[...]


def _stage_final(acc_ref, s_ref, dinv_ref, b3_ref, g3_ref, be3_ref, w_ref,
                 fb_ref, out_ref, *, n, c):
    a = acc_ref[0, :n, :] + acc_ref[1, :n, :]
    g = (a + s_ref[...]) * dinv_ref[...]
    g = g[:, :c] + b3_ref[...]
    m = jnp.mean(g, axis=0, keepdims=True)
    cv = g - m
    v = jnp.mean(cv * cv, axis=0, keepdims=True)
    r = jnp.maximum(cv * lax.rsqrt(v + 1e-5) * g3_ref[...] + be3_ref[...], 0.0)
    z = jnp.dot(r, w_ref[...], preferred_element_type=jnp.float32) + fb_ref[...]
    zm = jnp.max(z, axis=1, keepdims=True)
    e = jnp.exp(z - zm)
    out_ref[...] = z - (jnp.log(jnp.sum(e, axis=1, keepdims=True)) + zm)


def kernel(x, edge_index, fc1_W, fc1_b, W1, b1, g1, be1, W2, b2, g2, be2,
           W3, b3, g3, be3, fc2_W, fc2_b):
    n, d = x.shape
    c = fc2_W.shape[0]
    e = edge_index.shape[1]
    f3 = d

    epw = -(-e // _NW)
    nb = -(-epw // _B)
    etot = _NW * nb * _B
    nr = -(-(n + 1) // _B) * _B
    slab = nr // _NS

    src = edge_index[0]
    dst = edge_index[1]
    pad = etot - e
    srcp = jnp.concatenate([src, jnp.zeros((pad,), src.dtype)]).reshape(_NW, nb, _B)
    dstp = jnp.concatenate([dst, jnp.full((pad,), n, dst.dtype)]).reshape(_NW, nb, _B)

    zeros_fd = jnp.zeros((slab, _FD), jnp.float32)
    ones_b = jnp.ones((_B, _FD), jnp.float32)
    zeros_d = jnp.zeros((slab, d), jnp.float32)

    degp = _make_deg_kernel(nr, nb)(dstp, zeros_fd, ones_b)
    deg = degp[0, :n, 0] + degp[1, :n, 0] + 1.0
    dinv = lax.rsqrt(deg).reshape(n, 1)

    f32 = jnp.float32
    s1 = pl.pallas_call(
        _stage_a, out_shape=jax.ShapeDtypeStruct((n, d), f32),
    )(x, fc1_W, fc1_b, W1, dinv)

    scat_d = _make_scat_kernel(nr, nb, d)
    acc1 = scat_d(s1, srcp, dstp, zeros_d)
    s2 = pl.pallas_call(
        functools.partial(_stage_mid, n=n),
        out_shape=jax.ShapeDtypeStruct((n, d), f32),
    )(acc1, s1, dinv, b1, g1, be1, W2)

    acc2 = scat_d(s2, srcp, dstp, zeros_d)
    W3p = jnp.concatenate([W3, jnp.zeros((d, f3 - c), f32)], axis=1)
    s3 = pl.pallas_call(
        functools.partial(_stage_mid, n=n),
        out_shape=jax.ShapeDtypeStruct((n, f3), f32),
    )(acc2, s2, dinv, b2, g2, be2, W3p)

    acc3 = scat_d(s3, srcp, dstp, zeros_d)
    out = pl.pallas_call(
        functools.partial(_stage_final, n=n, c=c),
        out_shape=jax.ShapeDtypeStruct((n, c), f32),
    )(acc3, s3, dinv, b3, g3, be3, fc2_W, fc2_b)
    return out
